# baseline (speedup 1.0000x reference)
import functools

import jax
import jax.numpy as jnp
from jax import lax
from jax.experimental import pallas as pl
from jax.experimental.pallas import tpu as pltpu
from jax.experimental.pallas import tpu_sc as plsc

N = 10000
E = 320000
D = 128
NC = 2
NS = 16
NW = NC * NS
K = 80
EPW = E // NW
CH = EPW // K
CHP = CH // 2
RPT = N // NS

BN = 1000


def _sc_deg_body(dst_hbm, deg_out, didx_all, deg_v):
    c = lax.axis_index("c")
    s = lax.axis_index("s")
    wid = c * NS + s
    zvec = jnp.zeros((16,), jnp.float32)
    ovec = jnp.ones((16,), jnp.float32)

    def zr(r, _):
        deg_v[pl.ds(r * 16, 16)] = zvec
        return 0
    lax.fori_loop(0, N // 16, zr, 0)
    eoff = pl.multiple_of(wid * EPW, 8)
    pltpu.sync_copy(dst_hbm.at[pl.ds(eoff, EPW)], didx_all)

    def grp(j, _):
        idx = didx_all[pl.ds(j * 16, 16)]
        plsc.addupdate_scatter(deg_v, [idx], ovec)
        return 0
    lax.fori_loop(0, EPW // 16, grp, 0)
    pltpu.sync_copy(deg_v, deg_out.at[wid])


_sc_deg = pl.kernel(
    _sc_deg_body,
    out_type=[jax.ShapeDtypeStruct((NW, N), jnp.float32)],
    mesh=plsc.VectorSubcoreMesh(core_axis_name="c", subcore_axis_name="s"),
    scratch_types=[
        pltpu.VMEM((EPW,), jnp.int32),
        pltpu.VMEM((N,), jnp.float32),
    ],
    compiler_params=pltpu.CompilerParams(needs_layout_passes=False),
)


def _sc_agg_body(y_hbm, src_hbm, dst_hbm, zrs_hbm, agg_out, sidx_all,
                 didx_all, rows0, rows1, agg_sh, gs0, gs1, ss0, ss1):
    c = lax.axis_index("c")
    s = lax.axis_index("s")
    wid = c * NS + s
    base = s * RPT

    pltpu.sync_copy(zrs_hbm, agg_sh.at[pl.ds(base, RPT)])
    eoff = pl.multiple_of(wid * EPW, 8)
    pltpu.sync_copy(src_hbm.at[pl.ds(eoff, EPW)], sidx_all)
    pltpu.sync_copy(dst_hbm.at[pl.ds(eoff, EPW)], didx_all)
    plsc.subcore_barrier()

    rows = (rows0, rows1)
    gs = (gs0, gs1)
    ss = (ss0, ss1)
    for b in range(2):
        pltpu.async_copy(y_hbm.at[sidx_all.at[pl.ds(b * K, K)]],
                         rows[b], gs[b])

    def pair(i, _):
        for b in range(2):
            ci = i * 2 + b
            pltpu.make_async_copy(y_hbm.at[sidx_all.at[pl.ds(0, K)]],
                                  rows[b], gs[b]).wait()
            pltpu.async_copy(rows[b],
                             agg_sh.at[didx_all.at[pl.ds(ci * K, K)]],
                             ss[b], add=True)
        for b in range(2):
            ci = i * 2 + b + 2
            pltpu.make_async_copy(
                rows[b], agg_sh.at[didx_all.at[pl.ds(0, K)]], ss[b]).wait()

            @pl.when(ci < CH)
            def _():
                pltpu.async_copy(y_hbm.at[sidx_all.at[pl.ds(ci * K, K)]],
                                 rows[b], gs[b])
        return 0
    lax.fori_loop(0, CHP, pair, 0)
    if CH % 2:
        pltpu.make_async_copy(y_hbm.at[sidx_all.at[pl.ds(0, K)]],
                              rows[0], gs[0]).wait()
        pltpu.sync_copy(rows[0],
                        agg_sh.at[didx_all.at[pl.ds((CH - 1) * K, K)]],
                        add=True)
    plsc.subcore_barrier()

    pltpu.sync_copy(agg_sh.at[pl.ds(base, RPT)], agg_out.at[c, s])


_sc_agg = pl.kernel(
    _sc_agg_body,
    out_type=[jax.ShapeDtypeStruct((NC, NS, RPT, D), jnp.float32)],
    mesh=plsc.VectorSubcoreMesh(core_axis_name="c", subcore_axis_name="s"),
    scratch_types=[
        pltpu.VMEM((EPW,), jnp.int32),
        pltpu.VMEM((EPW,), jnp.int32),
        pltpu.VMEM((K, D), jnp.float32),
        pltpu.VMEM((K, D), jnp.float32),
        pltpu.VMEM_SHARED((N, D), jnp.float32),
        pltpu.SemaphoreType.DMA,
        pltpu.SemaphoreType.DMA,
        pltpu.SemaphoreType.DMA,
        pltpu.SemaphoreType.DMA,
    ],
)


def _tc_in_body(x_ref, wn_ref, ws_ref, bs_ref, y_ref, z_ref):
    xb = x_ref[...]
    dn = (((1,), (1,)), ((), ()))
    y_ref[...] = lax.dot_general(xb, wn_ref[...], dn,
                                 preferred_element_type=jnp.float32)
    z_ref[...] = lax.dot_general(xb, ws_ref[...], dn,
                                 preferred_element_type=jnp.float32) + bs_ref[...]


def _tc_in(x, Wn, Ws, bs):
    return pl.pallas_call(
        _tc_in_body,
        grid=(N // BN,),
        in_specs=[
            pl.BlockSpec((BN, D), lambda i: (i, 0)),
            pl.BlockSpec((D, D), lambda i: (0, 0)),
            pl.BlockSpec((D, D), lambda i: (0, 0)),
            pl.BlockSpec((1, D), lambda i: (0, 0)),
        ],
        out_specs=[
            pl.BlockSpec((BN, D), lambda i: (i, 0)),
            pl.BlockSpec((BN, D), lambda i: (i, 0)),
        ],
        out_shape=[
            jax.ShapeDtypeStruct((N, D), jnp.float32),
            jax.ShapeDtypeStruct((N, D), jnp.float32),
        ],
    )(x, Wn, Ws, bs.reshape(1, D))


def _tc_inv_body(degp_ref, o_ref):
    deg = lax.dot_general(degp_ref[...], jnp.ones((NW, 1), jnp.float32),
                          (((0,), (0,)), ((), ())),
                          preferred_element_type=jnp.float32)
    o_ref[...] = jnp.broadcast_to(1.0 / jnp.maximum(deg, 1.0), (N, 8))


def _tc_inv(degp):
    return pl.pallas_call(
        _tc_inv_body,
        in_specs=[pl.BlockSpec((NW, N), lambda: (0, 0))],
        out_specs=pl.BlockSpec((N, 8), lambda: (0, 0)),
        out_shape=jax.ShapeDtypeStruct((N, 8), jnp.float32),
    )(degp)


def _combine(aggp_ref, inv_ref, z_ref):
    agg = aggp_ref[0] + aggp_ref[1]
    return agg * inv_ref[:, 0:1] + z_ref[...]


def _tc_mid_body(aggp_ref, inv_ref, z_ref, g_ref, b_ref, wn_ref, ws_ref,
                 bs_ref, y_ref, z2_ref):
    a = _combine(aggp_ref, inv_ref, z_ref)
    m = jnp.mean(a, axis=1, keepdims=True)
    v = jnp.mean((a - m) * (a - m), axis=1, keepdims=True)
    h = (a - m) * lax.rsqrt(v + 1e-5) * g_ref[...] + b_ref[...]
    h = jnp.maximum(h, 0.0)
    dn = (((1,), (1,)), ((), ()))
    y_ref[...] = lax.dot_general(h, wn_ref[...], dn,
                                 preferred_element_type=jnp.float32)
    z2_ref[...] = lax.dot_general(h, ws_ref[...], dn,
                                  preferred_element_type=jnp.float32) + bs_ref[...]


def _tc_mid(aggp, inv8, z, g, b, Wn, Ws, bs):
    return pl.pallas_call(
        _tc_mid_body,
        grid=(N // BN,),
        in_specs=[
            pl.BlockSpec((NC, BN, D), lambda i: (0, i, 0)),
            pl.BlockSpec((BN, 8), lambda i: (i, 0)),
            pl.BlockSpec((BN, D), lambda i: (i, 0)),
            pl.BlockSpec((1, D), lambda i: (0, 0)),
            pl.BlockSpec((1, D), lambda i: (0, 0)),
            pl.BlockSpec((D, D), lambda i: (0, 0)),
            pl.BlockSpec((D, D), lambda i: (0, 0)),
            pl.BlockSpec((1, D), lambda i: (0, 0)),
        ],
        out_specs=[
            pl.BlockSpec((BN, D), lambda i: (i, 0)),
            pl.BlockSpec((BN, D), lambda i: (i, 0)),
        ],
        out_shape=[
            jax.ShapeDtypeStruct((N, D), jnp.float32),
            jax.ShapeDtypeStruct((N, D), jnp.float32),
        ],
    )(aggp, inv8, z, g.reshape(1, D), b.reshape(1, D), Wn, Ws,
      bs.reshape(1, D))


def _tc_fin_body(aggp_ref, inv_ref, z_ref, o_ref):
    a = _combine(aggp_ref, inv_ref, z_ref)
    nrm = jnp.sqrt(jnp.sum(a * a, axis=1, keepdims=True))
    o_ref[...] = a / jnp.maximum(nrm, 1e-12)


def _tc_fin(aggp, inv8, z):
    return pl.pallas_call(
        _tc_fin_body,
        grid=(N // BN,),
        in_specs=[
            pl.BlockSpec((NC, BN, D), lambda i: (0, i, 0)),
            pl.BlockSpec((BN, 8), lambda i: (i, 0)),
            pl.BlockSpec((BN, D), lambda i: (i, 0)),
        ],
        out_specs=pl.BlockSpec((BN, D), lambda i: (i, 0)),
        out_shape=jax.ShapeDtypeStruct((N, D), jnp.float32),
    )(aggp, inv8, z)


def _agg_part(y, src, dst, zrs):
    aggp, = _sc_agg(y, src, dst, zrs)
    return aggp.reshape(NC, N, D)


def kernel(x, edge_index, Wn1, Ws1, bs1, g1, b1, Wn2, Ws2, bs2, g2, b2,
           Wn3, Ws3, bs3):
    src = edge_index[0]
    dst = edge_index[1]

    zrs = jnp.zeros((RPT, D), jnp.float32)

    degp, = _sc_deg(dst)
    inv8 = _tc_inv(degp)
    y1, z1 = _tc_in(x, Wn1, Ws1, bs1)
    aggp1 = _agg_part(y1, src, dst, zrs)
    y2, z2 = _tc_mid(aggp1, inv8, z1, g1, b1, Wn2, Ws2, bs2)
    aggp2 = _agg_part(y2, src, dst, zrs)
    y3, z3 = _tc_mid(aggp2, inv8, z2, g2, b2, Wn3, Ws3, bs3)
    aggp3 = _agg_part(y3, src, dst, zrs)
    return _tc_fin(aggp3, inv8, z3)


# speedup vs baseline: 1.0200x; 1.0200x over previous
import functools

import jax
import jax.numpy as jnp
from jax import lax
from jax.experimental import pallas as pl
from jax.experimental.pallas import tpu as pltpu
from jax.experimental.pallas import tpu_sc as plsc

N = 10000
E = 320000
D = 128
NC = 2
NS = 16
NW = NC * NS
K = 80
EPW = E // NW
CH = EPW // K
CHP = CH // 2
RPT = N // NS

BN = 1000


def _sc_deg_body(dst_hbm, deg_out, didx_all, deg_v):
    c = lax.axis_index("c")
    s = lax.axis_index("s")
    wid = c * NS + s
    zvec = jnp.zeros((16,), jnp.float32)
    ovec = jnp.ones((16,), jnp.float32)

    def zr(r, _):
        deg_v[pl.ds(r * 16, 16)] = zvec
        return 0
    lax.fori_loop(0, N // 16, zr, 0)
    eoff = pl.multiple_of(wid * EPW, 8)
    pltpu.sync_copy(dst_hbm.at[pl.ds(eoff, EPW)], didx_all)

    def grp(j, _):
        idx = didx_all[pl.ds(j * 16, 16)]
        plsc.addupdate_scatter(deg_v, [idx], ovec)
        return 0
    lax.fori_loop(0, EPW // 16, grp, 0)
    pltpu.sync_copy(deg_v, deg_out.at[wid])


_sc_deg = pl.kernel(
    _sc_deg_body,
    out_type=[jax.ShapeDtypeStruct((NW, N), jnp.float32)],
    mesh=plsc.VectorSubcoreMesh(core_axis_name="c", subcore_axis_name="s"),
    scratch_types=[
        pltpu.VMEM((EPW,), jnp.int32),
        pltpu.VMEM((N,), jnp.float32),
    ],
    compiler_params=pltpu.CompilerParams(needs_layout_passes=False),
)


def _sc_agg_body(y_hbm, src_hbm, dst_hbm, agg_out, sidx_all,
                 didx_all, rows0, rows1, zbuf, agg_sh, gs0, gs1, ss0, ss1):
    c = lax.axis_index("c")
    s = lax.axis_index("s")
    wid = c * NS + s
    base = s * RPT
    zvec = jnp.zeros((16,), jnp.float32)

    def zrow(r, _):
        def zcol(j, _):
            zbuf[r, pl.ds(j * 16, 16)] = zvec
            return 0
        return lax.fori_loop(0, D // 16, zcol, 0)
    lax.fori_loop(0, 25, zrow, 0)

    for t in range(RPT // 25):
        pltpu.sync_copy(zbuf, agg_sh.at[pl.ds(base + t * 25, 25)])
    eoff = pl.multiple_of(wid * EPW, 8)
    pltpu.sync_copy(src_hbm.at[pl.ds(eoff, EPW)], sidx_all)
    pltpu.sync_copy(dst_hbm.at[pl.ds(eoff, EPW)], didx_all)
    plsc.subcore_barrier()

    rows = (rows0, rows1)
    gs = (gs0, gs1)
    ss = (ss0, ss1)
    for b in range(2):
        pltpu.async_copy(y_hbm.at[sidx_all.at[pl.ds(b * K, K)]],
                         rows[b], gs[b])

    def pair(i, _):
        for b in range(2):
            ci = i * 2 + b
            pltpu.make_async_copy(y_hbm.at[sidx_all.at[pl.ds(0, K)]],
                                  rows[b], gs[b]).wait()
            pltpu.async_copy(rows[b],
                             agg_sh.at[didx_all.at[pl.ds(ci * K, K)]],
                             ss[b], add=True)
        for b in range(2):
            ci = i * 2 + b + 2
            pltpu.make_async_copy(
                rows[b], agg_sh.at[didx_all.at[pl.ds(0, K)]], ss[b]).wait()

            @pl.when(ci < CH)
            def _():
                pltpu.async_copy(y_hbm.at[sidx_all.at[pl.ds(ci * K, K)]],
                                 rows[b], gs[b])
        return 0
    lax.fori_loop(0, CHP, pair, 0)
    if CH % 2:
        pltpu.make_async_copy(y_hbm.at[sidx_all.at[pl.ds(0, K)]],
                              rows[0], gs[0]).wait()
        pltpu.sync_copy(rows[0],
                        agg_sh.at[didx_all.at[pl.ds((CH - 1) * K, K)]],
                        add=True)
    plsc.subcore_barrier()

    pltpu.sync_copy(agg_sh.at[pl.ds(base, RPT)], agg_out.at[c, s])


_sc_agg = pl.kernel(
    _sc_agg_body,
    out_type=[jax.ShapeDtypeStruct((NC, NS, RPT, D), jnp.float32)],
    mesh=plsc.VectorSubcoreMesh(core_axis_name="c", subcore_axis_name="s"),
    scratch_types=[
        pltpu.VMEM((EPW,), jnp.int32),
        pltpu.VMEM((EPW,), jnp.int32),
        pltpu.VMEM((K, D), jnp.float32),
        pltpu.VMEM((K, D), jnp.float32),
        pltpu.VMEM((25, D), jnp.float32),
        pltpu.VMEM_SHARED((N, D), jnp.float32),
        pltpu.SemaphoreType.DMA,
        pltpu.SemaphoreType.DMA,
        pltpu.SemaphoreType.DMA,
        pltpu.SemaphoreType.DMA,
    ],
)


def _tc_in_body(x_ref, wn_ref, ws_ref, bs_ref, y_ref, z_ref):
    xb = x_ref[...]
    dn = (((1,), (1,)), ((), ()))
    y_ref[...] = lax.dot_general(xb, wn_ref[...], dn,
                                 preferred_element_type=jnp.float32)
    z_ref[...] = lax.dot_general(xb, ws_ref[...], dn,
                                 preferred_element_type=jnp.float32) + bs_ref[...]


def _tc_in(x, Wn, Ws, bs):
    return pl.pallas_call(
        _tc_in_body,
        grid=(N // BN,),
        in_specs=[
            pl.BlockSpec((BN, D), lambda i: (i, 0)),
            pl.BlockSpec((D, D), lambda i: (0, 0)),
            pl.BlockSpec((D, D), lambda i: (0, 0)),
            pl.BlockSpec((1, D), lambda i: (0, 0)),
        ],
        out_specs=[
            pl.BlockSpec((BN, D), lambda i: (i, 0)),
            pl.BlockSpec((BN, D), lambda i: (i, 0)),
        ],
        out_shape=[
            jax.ShapeDtypeStruct((N, D), jnp.float32),
            jax.ShapeDtypeStruct((N, D), jnp.float32),
        ],
    )(x, Wn, Ws, bs.reshape(1, D))


def _tc_inv_body(degp_ref, o_ref):
    deg = lax.dot_general(degp_ref[...], jnp.ones((NW, 1), jnp.float32),
                          (((0,), (0,)), ((), ())),
                          preferred_element_type=jnp.float32)
    o_ref[...] = jnp.broadcast_to(1.0 / jnp.maximum(deg, 1.0), (N, 8))


def _tc_inv(degp):
    return pl.pallas_call(
        _tc_inv_body,
        in_specs=[pl.BlockSpec((NW, N), lambda: (0, 0))],
        out_specs=pl.BlockSpec((N, 8), lambda: (0, 0)),
        out_shape=jax.ShapeDtypeStruct((N, 8), jnp.float32),
    )(degp)


def _combine(aggp_ref, inv_ref, z_ref):
    agg = aggp_ref[0] + aggp_ref[1]
    return agg * inv_ref[:, 0:1] + z_ref[...]


def _tc_mid_body(aggp_ref, inv_ref, z_ref, g_ref, b_ref, wn_ref, ws_ref,
                 bs_ref, y_ref, z2_ref):
    a = _combine(aggp_ref, inv_ref, z_ref)
    m = jnp.mean(a, axis=1, keepdims=True)
    v = jnp.mean((a - m) * (a - m), axis=1, keepdims=True)
    h = (a - m) * lax.rsqrt(v + 1e-5) * g_ref[...] + b_ref[...]
    h = jnp.maximum(h, 0.0)
    dn = (((1,), (1,)), ((), ()))
    y_ref[...] = lax.dot_general(h, wn_ref[...], dn,
                                 preferred_element_type=jnp.float32)
    z2_ref[...] = lax.dot_general(h, ws_ref[...], dn,
                                  preferred_element_type=jnp.float32) + bs_ref[...]


def _tc_mid(aggp, inv8, z, g, b, Wn, Ws, bs):
    return pl.pallas_call(
        _tc_mid_body,
        grid=(N // BN,),
        in_specs=[
            pl.BlockSpec((NC, BN, D), lambda i: (0, i, 0)),
            pl.BlockSpec((BN, 8), lambda i: (i, 0)),
            pl.BlockSpec((BN, D), lambda i: (i, 0)),
            pl.BlockSpec((1, D), lambda i: (0, 0)),
            pl.BlockSpec((1, D), lambda i: (0, 0)),
            pl.BlockSpec((D, D), lambda i: (0, 0)),
            pl.BlockSpec((D, D), lambda i: (0, 0)),
            pl.BlockSpec((1, D), lambda i: (0, 0)),
        ],
        out_specs=[
            pl.BlockSpec((BN, D), lambda i: (i, 0)),
            pl.BlockSpec((BN, D), lambda i: (i, 0)),
        ],
        out_shape=[
            jax.ShapeDtypeStruct((N, D), jnp.float32),
            jax.ShapeDtypeStruct((N, D), jnp.float32),
        ],
    )(aggp, inv8, z, g.reshape(1, D), b.reshape(1, D), Wn, Ws,
      bs.reshape(1, D))


def _tc_fin_body(aggp_ref, inv_ref, z_ref, o_ref):
    a = _combine(aggp_ref, inv_ref, z_ref)
    nrm = jnp.sqrt(jnp.sum(a * a, axis=1, keepdims=True))
    o_ref[...] = a / jnp.maximum(nrm, 1e-12)


def _tc_fin(aggp, inv8, z):
    return pl.pallas_call(
        _tc_fin_body,
        grid=(N // BN,),
        in_specs=[
            pl.BlockSpec((NC, BN, D), lambda i: (0, i, 0)),
            pl.BlockSpec((BN, 8), lambda i: (i, 0)),
            pl.BlockSpec((BN, D), lambda i: (i, 0)),
        ],
        out_specs=pl.BlockSpec((BN, D), lambda i: (i, 0)),
        out_shape=jax.ShapeDtypeStruct((N, D), jnp.float32),
    )(aggp, inv8, z)


def _agg_part(y, src, dst):
    aggp, = _sc_agg(y, src, dst)
    return aggp.reshape(NC, N, D)


def kernel(x, edge_index, Wn1, Ws1, bs1, g1, b1, Wn2, Ws2, bs2, g2, b2,
           Wn3, Ws3, bs3):
    src = edge_index[0]
    dst = edge_index[1]

    degp, = _sc_deg(dst)
    inv8 = _tc_inv(degp)
    y1, z1 = _tc_in(x, Wn1, Ws1, bs1)
    aggp1 = _agg_part(y1, src, dst)
    y2, z2 = _tc_mid(aggp1, inv8, z1, g1, b1, Wn2, Ws2, bs2)
    aggp2 = _agg_part(y2, src, dst)
    y3, z3 = _tc_mid(aggp2, inv8, z2, g2, b2, Wn3, Ws3, bs3)
    aggp3 = _agg_part(y3, src, dst)
    return _tc_fin(aggp3, inv8, z3)


# speedup vs baseline: 1.0208x; 1.0008x over previous
import jax
import jax.numpy as jnp
from jax import lax
from jax.experimental import pallas as pl
from jax.experimental.pallas import tpu as pltpu
from jax.experimental.pallas import tpu_sc as plsc

N = 10000
E = 320000
D = 128
NC = 2
NS = 16
NW = NC * NS
K = 80
EPW = E // NW
CH = EPW // K
CHP = CH // 2
RPT = N // NS

BN = 1000


def _sc_deg_body(dst_hbm, deg_out, didx_all, deg_v):
    c = lax.axis_index("c")
    s = lax.axis_index("s")
    wid = c * NS + s
    zvec = jnp.zeros((16,), jnp.float32)
    ovec = jnp.ones((16,), jnp.float32)

    def zr(r, _):
        deg_v[pl.ds(r * 16, 16)] = zvec
        return 0
    lax.fori_loop(0, N // 16, zr, 0)
    eoff = pl.multiple_of(wid * EPW, 8)
    pltpu.sync_copy(dst_hbm.at[pl.ds(eoff, EPW)], didx_all)

    def grp(j, _):
        idx = didx_all[pl.ds(j * 16, 16)]
        plsc.addupdate_scatter(deg_v, [idx], ovec)
        return 0
    lax.fori_loop(0, EPW // 16, grp, 0)
    pltpu.sync_copy(deg_v, deg_out.at[wid])


_sc_deg = pl.kernel(
    _sc_deg_body,
    out_type=[jax.ShapeDtypeStruct((NW, N), jnp.float32)],
    mesh=plsc.VectorSubcoreMesh(core_axis_name="c", subcore_axis_name="s"),
    scratch_types=[
        pltpu.VMEM((EPW,), jnp.int32),
        pltpu.VMEM((N,), jnp.float32),
    ],
    compiler_params=pltpu.CompilerParams(needs_layout_passes=False),
)


def _sc_agg_body(y_hbm, src_hbm, dst_hbm, agg_out, sidx_all,
                 didx_all, rows0, rows1, zbuf, agg_sh, gs0, gs1, ss0, ss1):
    c = lax.axis_index("c")
    s = lax.axis_index("s")
    wid = c * NS + s
    base = s * RPT
    zvec = jnp.zeros((16,), jnp.float32)

    def zrow(r, _):
        def zcol(j, _):
            zbuf[r, pl.ds(j * 16, 16)] = zvec
            return 0
        return lax.fori_loop(0, D // 16, zcol, 0)
    lax.fori_loop(0, 25, zrow, 0)

    for t in range(RPT // 25):
        pltpu.sync_copy(zbuf, agg_sh.at[pl.ds(base + t * 25, 25)])
    eoff = pl.multiple_of(wid * EPW, 8)
    pltpu.sync_copy(src_hbm.at[pl.ds(eoff, EPW)], sidx_all)
    pltpu.sync_copy(dst_hbm.at[pl.ds(eoff, EPW)], didx_all)
    plsc.subcore_barrier()

    rows = (rows0, rows1)
    gs = (gs0, gs1)
    ss = (ss0, ss1)
    for b in range(2):
        pltpu.async_copy(y_hbm.at[sidx_all.at[pl.ds(b * K, K)]],
                         rows[b], gs[b])

    def pair(i, _):
        for b in range(2):
            ci = i * 2 + b
            pltpu.make_async_copy(y_hbm.at[sidx_all.at[pl.ds(0, K)]],
                                  rows[b], gs[b]).wait()
            pltpu.async_copy(rows[b],
                             agg_sh.at[didx_all.at[pl.ds(ci * K, K)]],
                             ss[b], add=True)
        for b in range(2):
            ci = i * 2 + b + 2
            pltpu.make_async_copy(
                rows[b], agg_sh.at[didx_all.at[pl.ds(0, K)]], ss[b]).wait()

            @pl.when(ci < CH)
            def _():
                pltpu.async_copy(y_hbm.at[sidx_all.at[pl.ds(ci * K, K)]],
                                 rows[b], gs[b])
        return 0
    lax.fori_loop(0, CHP, pair, 0)
    if CH % 2:
        pltpu.make_async_copy(y_hbm.at[sidx_all.at[pl.ds(0, K)]],
                              rows[0], gs[0]).wait()
        pltpu.sync_copy(rows[0],
                        agg_sh.at[didx_all.at[pl.ds((CH - 1) * K, K)]],
                        add=True)
    plsc.subcore_barrier()

    pltpu.sync_copy(agg_sh.at[pl.ds(base, RPT)], agg_out.at[c, s])


_sc_agg = pl.kernel(
    _sc_agg_body,
    out_type=[jax.ShapeDtypeStruct((NC, NS, RPT, D), jnp.float32)],
    mesh=plsc.VectorSubcoreMesh(core_axis_name="c", subcore_axis_name="s"),
    scratch_types=[
        pltpu.VMEM((EPW,), jnp.int32),
        pltpu.VMEM((EPW,), jnp.int32),
        pltpu.VMEM((K, D), jnp.float32),
        pltpu.VMEM((K, D), jnp.float32),
        pltpu.VMEM((25, D), jnp.float32),
        pltpu.VMEM_SHARED((N, D), jnp.float32),
        pltpu.SemaphoreType.DMA,
        pltpu.SemaphoreType.DMA,
        pltpu.SemaphoreType.DMA,
        pltpu.SemaphoreType.DMA,
    ],
)


def _tc_in_body(x_ref, wn_ref, ws_ref, bs_ref, y_ref, z_ref):
    xb = x_ref[...]
    dn = (((1,), (1,)), ((), ()))
    y_ref[...] = lax.dot_general(xb, wn_ref[...], dn,
                                 preferred_element_type=jnp.float32)
    z_ref[...] = lax.dot_general(xb, ws_ref[...], dn,
                                 preferred_element_type=jnp.float32) + bs_ref[...]


def _tc_in(x, Wn, Ws, bs):
    return pl.pallas_call(
        _tc_in_body,
        grid=(N // BN,),
        in_specs=[
            pl.BlockSpec((BN, D), lambda i: (i, 0)),
            pl.BlockSpec((D, D), lambda i: (0, 0)),
            pl.BlockSpec((D, D), lambda i: (0, 0)),
            pl.BlockSpec((1, D), lambda i: (0, 0)),
        ],
        out_specs=[
            pl.BlockSpec((BN, D), lambda i: (i, 0)),
            pl.BlockSpec((BN, D), lambda i: (i, 0)),
        ],
        out_shape=[
            jax.ShapeDtypeStruct((N, D), jnp.float32),
            jax.ShapeDtypeStruct((N, D), jnp.float32),
        ],
    )(x, Wn, Ws, bs.reshape(1, D))


def _tc_inv_body(degp_ref, o_ref):
    deg = lax.dot_general(degp_ref[...], jnp.ones((NW, 1), jnp.float32),
                          (((0,), (0,)), ((), ())),
                          preferred_element_type=jnp.float32)
    o_ref[...] = jnp.broadcast_to(1.0 / jnp.maximum(deg, 1.0), (N, 8))


def _tc_inv(degp):
    return pl.pallas_call(
        _tc_inv_body,
        in_specs=[pl.BlockSpec((NW, N), lambda: (0, 0))],
        out_specs=pl.BlockSpec((N, 8), lambda: (0, 0)),
        out_shape=jax.ShapeDtypeStruct((N, 8), jnp.float32),
    )(degp)


def _combine(aggp_ref, inv_ref, z_ref):
    agg = aggp_ref[0] + aggp_ref[1]
    return agg * inv_ref[:, 0:1] + z_ref[...]


def _tc_mid_body(aggp_ref, inv_ref, z_ref, g_ref, b_ref, wn_ref, ws_ref,
                 bs_ref, y_ref, z2_ref):
    a = _combine(aggp_ref, inv_ref, z_ref)
    m = jnp.mean(a, axis=1, keepdims=True)
    v = jnp.mean((a - m) * (a - m), axis=1, keepdims=True)
    h = (a - m) * lax.rsqrt(v + 1e-5) * g_ref[...] + b_ref[...]
    h = jnp.maximum(h, 0.0)
    dn = (((1,), (1,)), ((), ()))
    y_ref[...] = lax.dot_general(h, wn_ref[...], dn,
                                 preferred_element_type=jnp.float32)
    z2_ref[...] = lax.dot_general(h, ws_ref[...], dn,
                                  preferred_element_type=jnp.float32) + bs_ref[...]


def _tc_mid(aggp, inv8, z, g, b, Wn, Ws, bs):
    return pl.pallas_call(
        _tc_mid_body,
        grid=(N // BN,),
        in_specs=[
            pl.BlockSpec((NC, BN, D), lambda i: (0, i, 0)),
            pl.BlockSpec((BN, 8), lambda i: (i, 0)),
            pl.BlockSpec((BN, D), lambda i: (i, 0)),
            pl.BlockSpec((1, D), lambda i: (0, 0)),
            pl.BlockSpec((1, D), lambda i: (0, 0)),
            pl.BlockSpec((D, D), lambda i: (0, 0)),
            pl.BlockSpec((D, D), lambda i: (0, 0)),
            pl.BlockSpec((1, D), lambda i: (0, 0)),
        ],
        out_specs=[
            pl.BlockSpec((BN, D), lambda i: (i, 0)),
            pl.BlockSpec((BN, D), lambda i: (i, 0)),
        ],
        out_shape=[
            jax.ShapeDtypeStruct((N, D), jnp.float32),
            jax.ShapeDtypeStruct((N, D), jnp.float32),
        ],
    )(aggp, inv8, z, g.reshape(1, D), b.reshape(1, D), Wn, Ws,
      bs.reshape(1, D))


def _tc_fin_body(aggp_ref, inv_ref, z_ref, o_ref):
    a = _combine(aggp_ref, inv_ref, z_ref)
    nrm = jnp.sqrt(jnp.sum(a * a, axis=1, keepdims=True))
    o_ref[...] = a / jnp.maximum(nrm, 1e-12)


def _tc_fin(aggp, inv8, z):
    return pl.pallas_call(
        _tc_fin_body,
        grid=(N // BN,),
        in_specs=[
            pl.BlockSpec((NC, BN, D), lambda i: (0, i, 0)),
            pl.BlockSpec((BN, 8), lambda i: (i, 0)),
            pl.BlockSpec((BN, D), lambda i: (i, 0)),
        ],
        out_specs=pl.BlockSpec((BN, D), lambda i: (i, 0)),
        out_shape=jax.ShapeDtypeStruct((N, D), jnp.float32),
    )(aggp, inv8, z)


def _agg_part(y, src, dst):
    aggp, = _sc_agg(y, src, dst)
    return aggp.reshape(NC, N, D)


def kernel(x, edge_index, Wn1, Ws1, bs1, g1, b1, Wn2, Ws2, bs2, g2, b2,
           Wn3, Ws3, bs3):
    src = edge_index[0]
    dst = edge_index[1]

    degp, = _sc_deg(dst)
    inv8 = _tc_inv(degp)
    y1, z1 = _tc_in(x, Wn1, Ws1, bs1)
    aggp1 = _agg_part(y1, src, dst)
    y2, z2 = _tc_mid(aggp1, inv8, z1, g1, b1, Wn2, Ws2, bs2)
    aggp2 = _agg_part(y2, src, dst)
    y3, z3 = _tc_mid(aggp2, inv8, z2, g2, b2, Wn3, Ws3, bs3)
    aggp3 = _agg_part(y3, src, dst)
    return _tc_fin(aggp3, inv8, z3)
